# baseline (device time: 80645 ns/iter reference)
import jax
import jax.numpy as jnp
from jax import lax
from jax.experimental import pallas as pl
from jax.experimental.pallas import tpu as pltpu

N_DEV = 8
M_BLK = 512
SZ = (2048, 1024, 512)
HALF = (1024, 512, 256)
SCRATCH_OFF = (0, 2048, 3072)
SCRATCH_ROWS = 3584

GROUPS = (
    {"c0": 0, "ng": 640, "order": ("x", "y", "z")},
    {"c0": 640, "ng": 640, "order": ("y", "z", "x")},
    {"c0": 1280, "ng": 768, "order": ("z", "x", "y")},
)


def _slot_chunk(order, s):
    k = ((s >> 2) & 1, (s >> 1) & 1, s & 1)
    bits = dict(zip(order, k))
    return 4 * bits["z"] + 2 * bits["y"] + (bits["x"] ^ bits["y"])


def kernel(x, w_mat, scale_x, scale_w):
    m_glob, k_sh = x.shape
    _, n = w_mat.shape

    def body(x_ref, w_ref, sx_ref, sw_ref, out_ref,
             buf0, buf1, buf2, rcv0, rcv1, rcv2, x_bf, w_bf,
             send_sems, recv_sems):
        bufs = (buf0, buf1, buf2)
        rcvs = (rcv0, rcv1, rcv2)

        my = lax.axis_index("i")
        zc = my // 4
        r = my % 4
        yc = r // 2
        xc = jnp.where((r == 1) | (r == 2), 1, 0)

        def dev_id(cx, cy, cz):
            return 4 * cz + 2 * cy + (cx + cy) % 2

        coord = {"x": xc, "y": yc, "z": zc}
        nbr = {
            "x": dev_id(1 - xc, yc, zc),
            "y": dev_id(xc, 1 - yc, zc),
            "z": dev_id(xc, yc, 1 - zc),
        }

        barrier_sem = pltpu.get_barrier_semaphore()
        for a in ("x", "y", "z"):
            pl.semaphore_signal(
                barrier_sem, inc=1,
                device_id=(nbr[a],), device_id_type=pl.DeviceIdType.MESH,
            )
        pl.semaphore_wait(barrier_sem, 3)

        def make_rdma(g, k, sub, send_off):
            h = HALF[k]
            return pltpu.make_async_remote_copy(
                src_ref=bufs[g].at[pl.ds(send_off + sub * h, h)],
                dst_ref=rcvs[g].at[pl.ds(SCRATCH_OFF[k] + sub * h, h)],
                send_sem=send_sems.at[g, k, sub],
                recv_sem=recv_sems.at[g, k, sub],
                device_id=(nbr[GROUPS[g]["order"][k]],),
                device_id_type=pl.DeviceIdType.MESH,
            )

        off = [0, 0, 0]
        send_offs = [[0] * 3 for _ in range(3)]

        x_bf[:, :] = x_ref[:, :].astype(jnp.bfloat16)
        w_bf[:, :] = w_ref[:, :].astype(jnp.bfloat16)

        def gemm_half(g, k1, j):
            grp = GROUPS[g]
            c0, ng = grp["c0"], grp["ng"]
            bits = dict(zip(grp["order"], (k1, (j >> 1) & 1, j & 1)))
            cid = dev_id(bits["x"], bits["y"], bits["z"])
            part = lax.dot_general(
                x_bf[pl.ds(cid * M_BLK, M_BLK), :],
                w_bf[:, c0:c0 + ng],
                (((1,), (0,)), ((), ())),
                preferred_element_type=jnp.float32,
            )
            bufs[g][pl.ds((4 * k1 + j) * M_BLK, M_BLK), :] = (
                part.astype(jnp.bfloat16)
            )

        for g in (2, 0, 1):
            b = coord[GROUPS[g]["order"][0]]
            for j in range(4):
                gemm_half(g, 1 - b, j)
            so = (1 - b) * SZ[0]
            send_offs[g][0] = so
            off[g] = b * SZ[0]
            for sub in range(2):
                make_rdma(g, 0, sub, so).start()
        for g in (2, 0, 1):
            b = coord[GROUPS[g]["order"][0]]
            for j in range(4):
                gemm_half(g, b, j)

        scale = sx_ref[0] * sw_ref[0]
        for k in range(3):
            h = HALF[k]
            for sub in range(2):
                for g in range(3):
                    grp = GROUPS[g]
                    make_rdma(g, k, sub, send_offs[g][k]).wait_recv()
                    rows = pl.ds(off[g] + sub * h, h)
                    srows = pl.ds(SCRATCH_OFF[k] + sub * h, h)
                    if k < 2:
                        bufs[g][rows, :] = (
                            bufs[g][rows, :].astype(jnp.float32)
                            + rcvs[g][srows, :].astype(jnp.float32)
                        ).astype(jnp.bfloat16)
                        if sub == 1:
                            b = coord[grp["order"][k + 1]]
                            so = off[g] + (1 - b) * SZ[k + 1]
                            send_offs[g][k + 1] = so
                            off[g] = off[g] + b * SZ[k + 1]
                            for s2 in range(2):
                                make_rdma(g, k + 1, s2, so).start()
                    else:
                        c0, ng = grp["c0"], grp["ng"]
                        acc = (
                            bufs[g][rows, :].astype(jnp.float32)
                            + rcvs[g][srows, :].astype(jnp.float32)
                        )
                        out_ref[pl.ds(sub * h, h), c0:c0 + ng] = jnp.maximum(
                            acc * scale, 0.0
                        )

        for g in range(3):
            for k in range(3):
                for sub in range(2):
                    make_rdma(g, k, sub, send_offs[g][k]).wait_send()

    return pl.pallas_call(
        body,
        out_shape=jax.ShapeDtypeStruct((M_BLK, n), jnp.float32),
        in_specs=[
            pl.BlockSpec(memory_space=pltpu.VMEM),
            pl.BlockSpec(memory_space=pltpu.VMEM),
            pl.BlockSpec(memory_space=pltpu.SMEM),
            pl.BlockSpec(memory_space=pltpu.SMEM),
        ],
        out_specs=pl.BlockSpec(memory_space=pltpu.VMEM),
        scratch_shapes=[
            pltpu.VMEM((m_glob, GROUPS[0]["ng"]), jnp.bfloat16),
            pltpu.VMEM((m_glob, GROUPS[1]["ng"]), jnp.bfloat16),
            pltpu.VMEM((m_glob, GROUPS[2]["ng"]), jnp.bfloat16),
            pltpu.VMEM((SCRATCH_ROWS, GROUPS[0]["ng"]), jnp.bfloat16),
            pltpu.VMEM((SCRATCH_ROWS, GROUPS[1]["ng"]), jnp.bfloat16),
            pltpu.VMEM((SCRATCH_ROWS, GROUPS[2]["ng"]), jnp.bfloat16),
            pltpu.VMEM((m_glob, k_sh), jnp.bfloat16),
            pltpu.VMEM((k_sh, n), jnp.bfloat16),
            pltpu.SemaphoreType.DMA((3, 3, 2)),
            pltpu.SemaphoreType.DMA((3, 3, 2)),
        ],
        compiler_params=pltpu.CompilerParams(
            collective_id=0,
            vmem_limit_bytes=50 * 1024 * 1024,
        ),
    )(x, w_mat, scale_x, scale_w)


# device time: 79655 ns/iter; 1.0124x vs baseline; 1.0124x over previous
import jax
import jax.numpy as jnp
from jax import lax
from jax.experimental import pallas as pl
from jax.experimental.pallas import tpu as pltpu

N_DEV = 8
M_BLK = 512
SZ = (2048, 1024, 512)
SUBS = (4, 2, 4)
HALF = (512, 512, 128)
SCRATCH_OFF = (0, 2048, 3072)
SCRATCH_ROWS = 3584

GROUPS = (
    {"c0": 0, "ng": 640, "order": ("x", "y", "z")},
    {"c0": 640, "ng": 640, "order": ("y", "z", "x")},
    {"c0": 1280, "ng": 768, "order": ("z", "x", "y")},
)


def _slot_chunk(order, s):
    k = ((s >> 2) & 1, (s >> 1) & 1, s & 1)
    bits = dict(zip(order, k))
    return 4 * bits["z"] + 2 * bits["y"] + (bits["x"] ^ bits["y"])


def kernel(x, w_mat, scale_x, scale_w):
    m_glob, k_sh = x.shape
    _, n = w_mat.shape

    def body(x_ref, w_ref, sx_ref, sw_ref, out_ref,
             buf0, buf1, buf2, rcv0, rcv1, rcv2,
             send_sems, recv_sems):
        bufs = (buf0, buf1, buf2)
        rcvs = (rcv0, rcv1, rcv2)

        my = lax.axis_index("i")
        zc = my // 4
        r = my % 4
        yc = r // 2
        xc = jnp.where((r == 1) | (r == 2), 1, 0)

        def dev_id(cx, cy, cz):
            return 4 * cz + 2 * cy + (cx + cy) % 2

        coord = {"x": xc, "y": yc, "z": zc}
        nbr = {
            "x": dev_id(1 - xc, yc, zc),
            "y": dev_id(xc, 1 - yc, zc),
            "z": dev_id(xc, yc, 1 - zc),
        }

        barrier_sem = pltpu.get_barrier_semaphore()
        for a in ("x", "y", "z"):
            pl.semaphore_signal(
                barrier_sem, inc=1,
                device_id=(nbr[a],), device_id_type=pl.DeviceIdType.MESH,
            )
        pl.semaphore_wait(barrier_sem, 3)

        def make_rdma(g, k, sub, send_off):
            h = HALF[k]
            return pltpu.make_async_remote_copy(
                src_ref=bufs[g].at[pl.ds(send_off + sub * h, h)],
                dst_ref=rcvs[g].at[pl.ds(SCRATCH_OFF[k] + sub * h, h)],
                send_sem=send_sems.at[g, k, sub],
                recv_sem=recv_sems.at[g, k, sub],
                device_id=(nbr[GROUPS[g]["order"][k]],),
                device_id_type=pl.DeviceIdType.MESH,
            )

        off = [0, 0, 0]
        send_offs = [[0] * 3 for _ in range(3)]

        def gemm_half(g, k1, j):
            grp = GROUPS[g]
            c0, ng = grp["c0"], grp["ng"]
            bits = dict(zip(grp["order"], (k1, (j >> 1) & 1, j & 1)))
            cid = dev_id(bits["x"], bits["y"], bits["z"])
            part = lax.dot_general(
                x_ref[pl.ds(cid * M_BLK, M_BLK), :],
                w_ref[:, c0:c0 + ng],
                (((1,), (0,)), ((), ())),
                preferred_element_type=jnp.int32,
            )
            bufs[g][pl.ds((4 * k1 + j) * M_BLK, M_BLK), :] = (
                part.astype(jnp.bfloat16)
            )

        for g in (2, 0, 1):
            b = coord[GROUPS[g]["order"][0]]
            for j in range(4):
                gemm_half(g, 1 - b, j)
            so = (1 - b) * SZ[0]
            send_offs[g][0] = so
            off[g] = b * SZ[0]
            for sub in range(SUBS[0]):
                make_rdma(g, 0, sub, so).start()
        for g in (2, 0, 1):
            b = coord[GROUPS[g]["order"][0]]
            for j in range(4):
                gemm_half(g, b, j)

        scale = sx_ref[0] * sw_ref[0]
        for k in range(3):
            h = HALF[k]
            for sub in range(SUBS[k]):
                for g in range(3):
                    grp = GROUPS[g]
                    make_rdma(g, k, sub, send_offs[g][k]).wait_recv()
                    rows = pl.ds(off[g] + sub * h, h)
                    srows = pl.ds(SCRATCH_OFF[k] + sub * h, h)
                    if k < 2:
                        bufs[g][rows, :] = (
                            bufs[g][rows, :].astype(jnp.float32)
                            + rcvs[g][srows, :].astype(jnp.float32)
                        ).astype(jnp.bfloat16)
                        if sub == SUBS[k] - 1:
                            b = coord[grp["order"][k + 1]]
                            so = off[g] + (1 - b) * SZ[k + 1]
                            send_offs[g][k + 1] = so
                            off[g] = off[g] + b * SZ[k + 1]
                            for s2 in range(SUBS[k + 1]):
                                make_rdma(g, k + 1, s2, so).start()
                    else:
                        c0, ng = grp["c0"], grp["ng"]
                        acc = (
                            bufs[g][rows, :].astype(jnp.float32)
                            + rcvs[g][srows, :].astype(jnp.float32)
                        )
                        out_ref[pl.ds(sub * h, h), c0:c0 + ng] = jnp.maximum(
                            acc * scale, 0.0
                        )

        for g in range(3):
            for k in range(3):
                for sub in range(SUBS[k]):
                    make_rdma(g, k, sub, send_offs[g][k]).wait_send()

    return pl.pallas_call(
        body,
        out_shape=jax.ShapeDtypeStruct((M_BLK, n), jnp.float32),
        in_specs=[
            pl.BlockSpec(memory_space=pltpu.VMEM),
            pl.BlockSpec(memory_space=pltpu.VMEM),
            pl.BlockSpec(memory_space=pltpu.SMEM),
            pl.BlockSpec(memory_space=pltpu.SMEM),
        ],
        out_specs=pl.BlockSpec(memory_space=pltpu.VMEM),
        scratch_shapes=[
            pltpu.VMEM((m_glob, GROUPS[0]["ng"]), jnp.bfloat16),
            pltpu.VMEM((m_glob, GROUPS[1]["ng"]), jnp.bfloat16),
            pltpu.VMEM((m_glob, GROUPS[2]["ng"]), jnp.bfloat16),
            pltpu.VMEM((SCRATCH_ROWS, GROUPS[0]["ng"]), jnp.bfloat16),
            pltpu.VMEM((SCRATCH_ROWS, GROUPS[1]["ng"]), jnp.bfloat16),
            pltpu.VMEM((SCRATCH_ROWS, GROUPS[2]["ng"]), jnp.bfloat16),
            pltpu.SemaphoreType.DMA((3, 3, 4)),
            pltpu.SemaphoreType.DMA((3, 3, 4)),
        ],
        compiler_params=pltpu.CompilerParams(
            collective_id=0,
            vmem_limit_bytes=50 * 1024 * 1024,
        ),
    )(x, w_mat, scale_x, scale_w)
